# baseline (speedup 1.0000x reference)
import functools

import jax
import jax.numpy as jnp
from jax import lax
from jax.experimental import pallas as pl
from jax.experimental.pallas import tpu as pltpu
from jax.experimental.pallas import tpu_sc as plsc

D = 64
NC = 2
NS = 16
NW = NC * NS
CHUNK = 128
K = 2
NSETS = 4
GROUP_ROWS = K * CHUNK


@functools.cache
def _build(total):
    per_w = total // NW
    nchunk = per_w // CHUNK
    ngroups = nchunk // K
    nround = ngroups // NSETS
    mesh = plsc.VectorSubcoreMesh(core_axis_name="c", subcore_axis_name="s")

    @functools.partial(
        pl.kernel,
        mesh=mesh,
        out_type=jax.ShapeDtypeStruct((NW, ngroups, GROUP_ROWS, D), jnp.float32),
        scratch_types=[
            pltpu.VMEM((nchunk, CHUNK), jnp.int32),
        ] + [pltpu.VMEM((GROUP_ROWS, D), jnp.float32)] * NSETS
          + [pltpu.SemaphoreType.DMA] * (2 * NSETS),
        compiler_params=pltpu.CompilerParams(use_tc_tiling_on_sc=False),
    )
    def gather_kernel(idx_hbm, table_hbm, out_hbm, idx_v, *bufsem):
        bufs = bufsem[:NSETS]
        sgs = bufsem[NSETS:2 * NSETS]
        sss = bufsem[2 * NSETS:]
        wid = lax.axis_index("s") * NC + lax.axis_index("c")
        pltpu.sync_copy(idx_hbm.at[wid], idx_v)

        def round_body(p, carry):
            copies = []
            for s in range(NSETS):
                g = NSETS * p + s
                @pl.when(p > 0)
                def _():
                    pltpu.make_async_copy(
                        bufs[s], out_hbm.at[wid, g - NSETS], sss[s]).wait()
                copies.append([
                    pltpu.async_copy(
                        table_hbm.at[idx_v.at[g * K + b]],
                        bufs[s].at[pl.ds(b * CHUNK, CHUNK)], sgs[s])
                    for b in range(K)
                ])
            for s in range(NSETS):
                g = NSETS * p + s
                for c in copies[s]:
                    c.wait()
                pltpu.async_copy(bufs[s], out_hbm.at[wid, g], sss[s])
            return carry

        lax.fori_loop(0, nround, round_body, 0)
        for s in range(NSETS):
            pltpu.make_async_copy(
                bufs[s], out_hbm.at[wid, ngroups - NSETS + s], sss[s]).wait()

    return gather_kernel


TBLK = 1024


@functools.cache
def _build_transpose(v, d):
    nblk = (v + TBLK - 1) // TBLK
    half = TBLK // 2

    def body(wt_ref, out_ref):
        wt = wt_ref[...]
        out_ref[...] = jnp.concatenate(
            [jnp.transpose(wt[:, :half], (1, 0)),
             jnp.transpose(wt[:, half:], (1, 0))], axis=1)

    return pl.pallas_call(
        body,
        grid=(nblk,),
        in_specs=[pl.BlockSpec((d, TBLK), lambda i: (0, i))],
        out_specs=pl.BlockSpec((half, 2 * d), lambda i: (i, 0)),
        out_shape=jax.ShapeDtypeStruct((nblk * half, 2 * d), jnp.float32),
    ), nblk * TBLK


def kernel(inp, weight):
    total = inp.shape[0] * inp.shape[1]
    nchunk = total // NW // CHUNK
    v, d = weight.shape
    xpose, vpad = _build_transpose(v, d)
    table = xpose(weight.T).reshape(vpad, d)
    t = inp.astype(jnp.int32)
    tj = t % TBLK
    ridx = (t - tj) + 2 * (tj % (TBLK // 2)) + tj // (TBLK // 2)
    idx = ridx.reshape(NW, nchunk, CHUNK)
    out = _build(total)(idx, table)
    return out.reshape(inp.shape[0], inp.shape[1], weight.shape[1])


# speedup vs baseline: 1.3657x; 1.3657x over previous
import functools

import jax
import jax.numpy as jnp
from jax import lax
from jax.experimental import pallas as pl
from jax.experimental.pallas import tpu as pltpu
from jax.experimental.pallas import tpu_sc as plsc

D = 64
NC = 2
NS = 16
NW = NC * NS
CHUNK = 128
K = 2
NSETS = 4
GROUP_ROWS = K * CHUNK


@functools.cache
def _build(total):
    per_w = total // NW
    nchunk = per_w // CHUNK
    ngroups = nchunk // K
    nround = ngroups // NSETS
    mesh = plsc.VectorSubcoreMesh(core_axis_name="c", subcore_axis_name="s")

    @functools.partial(
        pl.kernel,
        mesh=mesh,
        out_type=jax.ShapeDtypeStruct((NW, ngroups, GROUP_ROWS, D), jnp.float32),
        scratch_types=[
            pltpu.VMEM((nchunk, CHUNK), jnp.int32),
        ] + [pltpu.VMEM((GROUP_ROWS, D), jnp.float32)] * NSETS
          + [pltpu.SemaphoreType.DMA] * (2 * NSETS),
        compiler_params=pltpu.CompilerParams(use_tc_tiling_on_sc=False),
    )
    def gather_kernel(idx_hbm, table_hbm, out_hbm, idx_v, *bufsem):
        bufs = bufsem[:NSETS]
        sgs = bufsem[NSETS:2 * NSETS]
        sss = bufsem[2 * NSETS:]
        wid = lax.axis_index("s") * NC + lax.axis_index("c")
        pltpu.sync_copy(idx_hbm.at[wid], idx_v)

        def round_body(p, carry):
            copies = []
            for s in range(NSETS):
                g = NSETS * p + s
                @pl.when(p > 0)
                def _():
                    pltpu.make_async_copy(
                        bufs[s], out_hbm.at[wid, g - NSETS], sss[s]).wait()
                copies.append([
                    pltpu.async_copy(
                        table_hbm.at[idx_v.at[g * K + b]],
                        bufs[s].at[pl.ds(b * CHUNK, CHUNK)], sgs[s])
                    for b in range(K)
                ])
            for s in range(NSETS):
                g = NSETS * p + s
                for c in copies[s]:
                    c.wait()
                pltpu.async_copy(bufs[s], out_hbm.at[wid, g], sss[s])
            return carry

        lax.fori_loop(0, nround, round_body, 0)
        for s in range(NSETS):
            pltpu.make_async_copy(
                bufs[s], out_hbm.at[wid, ngroups - NSETS + s], sss[s]).wait()

    return gather_kernel


TBLK = 4096


@functools.cache
def _build_transpose(v, d):
    nblk = (v + TBLK - 1) // TBLK
    half = TBLK // 2

    def body(wt_ref, out_ref):
        wt = wt_ref[...]
        out_ref[...] = jnp.concatenate(
            [jnp.transpose(wt[:, :half], (1, 0)),
             jnp.transpose(wt[:, half:], (1, 0))], axis=1)

    return pl.pallas_call(
        body,
        grid=(nblk,),
        in_specs=[pl.BlockSpec((d, TBLK), lambda i: (0, i))],
        out_specs=pl.BlockSpec((half, 2 * d), lambda i: (i, 0)),
        out_shape=jax.ShapeDtypeStruct((nblk * half, 2 * d), jnp.float32),
    ), nblk * TBLK


def kernel(inp, weight):
    total = inp.shape[0] * inp.shape[1]
    nchunk = total // NW // CHUNK
    v, d = weight.shape
    xpose, vpad = _build_transpose(v, d)
    table = xpose(weight.T).reshape(vpad, d)
    t = inp.astype(jnp.int32)
    tj = t % TBLK
    ridx = (t - tj) + 2 * (tj % (TBLK // 2)) + tj // (TBLK // 2)
    idx = ridx.reshape(NW, nchunk, CHUNK)
    out = _build(total)(idx, table)
    return out.reshape(inp.shape[0], inp.shape[1], weight.shape[1])


# speedup vs baseline: 1.4677x; 1.0747x over previous
import functools

import jax
import jax.numpy as jnp
from jax import lax
from jax.experimental import pallas as pl
from jax.experimental.pallas import tpu as pltpu
from jax.experimental.pallas import tpu_sc as plsc

D = 64
NC = 2
NS = 16
NW = NC * NS
CHUNK = 128
K = 2
NSETS = 4
GROUP_ROWS = K * CHUNK


@functools.cache
def _build(total):
    per_w = total // NW
    nchunk = per_w // CHUNK
    ngroups = nchunk // K
    nround = ngroups // NSETS
    mesh = plsc.VectorSubcoreMesh(core_axis_name="c", subcore_axis_name="s")

    @functools.partial(
        pl.kernel,
        mesh=mesh,
        out_type=jax.ShapeDtypeStruct((NW, ngroups, GROUP_ROWS, D), jnp.float32),
        scratch_types=[
            pltpu.VMEM((nchunk, CHUNK), jnp.int32),
        ] + [pltpu.VMEM((GROUP_ROWS, D), jnp.float32)] * NSETS
          + [pltpu.SemaphoreType.DMA] * (2 * NSETS),
        compiler_params=pltpu.CompilerParams(use_tc_tiling_on_sc=False),
    )
    def gather_kernel(idx_hbm, table_hbm, out_hbm, idx_v, *bufsem):
        bufs = bufsem[:NSETS]
        sgs = bufsem[NSETS:2 * NSETS]
        sss = bufsem[2 * NSETS:]
        wid = lax.axis_index("s") * NC + lax.axis_index("c")
        pltpu.sync_copy(idx_hbm.at[wid], idx_v)

        def round_body(p, carry):
            copies = []
            for s in range(NSETS):
                g = NSETS * p + s
                @pl.when(p > 0)
                def _():
                    pltpu.make_async_copy(
                        bufs[s], out_hbm.at[wid, g - NSETS], sss[s]).wait()
                copies.append([
                    pltpu.async_copy(
                        table_hbm.at[idx_v.at[g * K + b]],
                        bufs[s].at[pl.ds(b * CHUNK, CHUNK)], sgs[s])
                    for b in range(K)
                ])
            for s in range(NSETS):
                g = NSETS * p + s
                for c in copies[s]:
                    c.wait()
                pltpu.async_copy(bufs[s], out_hbm.at[wid, g], sss[s])
            return carry

        lax.fori_loop(0, nround, round_body, 0)
        for s in range(NSETS):
            pltpu.make_async_copy(
                bufs[s], out_hbm.at[wid, ngroups - NSETS + s], sss[s]).wait()

    return gather_kernel


TBLK = 8192


@functools.cache
def _build_transpose(v, d):
    nblk = (v + TBLK - 1) // TBLK
    half = TBLK // 2

    def body(wt_ref, out_ref):
        wt = wt_ref[...]
        out_ref[...] = jnp.concatenate(
            [jnp.transpose(wt[:, :half], (1, 0)),
             jnp.transpose(wt[:, half:], (1, 0))], axis=1)

    return pl.pallas_call(
        body,
        grid=(nblk,),
        in_specs=[pl.BlockSpec((d, TBLK), lambda i: (0, i))],
        out_specs=pl.BlockSpec((half, 2 * d), lambda i: (i, 0)),
        out_shape=jax.ShapeDtypeStruct((nblk * half, 2 * d), jnp.float32),
    ), nblk * TBLK


def kernel(inp, weight):
    total = inp.shape[0] * inp.shape[1]
    nchunk = total // NW // CHUNK
    v, d = weight.shape
    xpose, vpad = _build_transpose(v, d)
    table = xpose(weight.T).reshape(vpad, d)
    t = inp.astype(jnp.int32)
    tj = t % TBLK
    ridx = (t - tj) + 2 * (tj % (TBLK // 2)) + tj // (TBLK // 2)
    idx = ridx.reshape(NW, nchunk, CHUNK)
    out = _build(total)(idx, table)
    return out.reshape(inp.shape[0], inp.shape[1], weight.shape[1])


# speedup vs baseline: 1.5204x; 1.0359x over previous
import functools

import jax
import jax.numpy as jnp
from jax import lax
from jax.experimental import pallas as pl
from jax.experimental.pallas import tpu as pltpu
from jax.experimental.pallas import tpu_sc as plsc

D = 64
NC = 2
NS = 16
NW = NC * NS
CHUNK = 128
K = 2
NSETS = 4
GROUP_ROWS = K * CHUNK


@functools.cache
def _build(total):
    per_w = total // NW
    nchunk = per_w // CHUNK
    ngroups = nchunk // K
    nround = ngroups // NSETS
    mesh = plsc.VectorSubcoreMesh(core_axis_name="c", subcore_axis_name="s")

    @functools.partial(
        pl.kernel,
        mesh=mesh,
        out_type=jax.ShapeDtypeStruct((NW, ngroups, GROUP_ROWS, D), jnp.float32),
        scratch_types=[
            pltpu.VMEM((nchunk, CHUNK), jnp.int32),
        ] + [pltpu.VMEM((GROUP_ROWS, D), jnp.float32)] * NSETS
          + [pltpu.SemaphoreType.DMA] * (2 * NSETS),
        compiler_params=pltpu.CompilerParams(use_tc_tiling_on_sc=False),
    )
    def gather_kernel(idx_hbm, table_hbm, out_hbm, idx_v, *bufsem):
        bufs = bufsem[:NSETS]
        sgs = bufsem[NSETS:2 * NSETS]
        sss = bufsem[2 * NSETS:]
        wid = lax.axis_index("s") * NC + lax.axis_index("c")
        pltpu.sync_copy(idx_hbm.at[wid], idx_v)

        def round_body(p, carry):
            copies = []
            for s in range(NSETS):
                g = NSETS * p + s
                @pl.when(p > 0)
                def _():
                    pltpu.make_async_copy(
                        bufs[s], out_hbm.at[wid, g - NSETS], sss[s]).wait()
                copies.append([
                    pltpu.async_copy(
                        table_hbm.at[idx_v.at[g * K + b]],
                        bufs[s].at[pl.ds(b * CHUNK, CHUNK)], sgs[s])
                    for b in range(K)
                ])
            for s in range(NSETS):
                g = NSETS * p + s
                for c in copies[s]:
                    c.wait()
                pltpu.async_copy(bufs[s], out_hbm.at[wid, g], sss[s])
            return carry

        lax.fori_loop(0, nround, round_body, 0)
        for s in range(NSETS):
            pltpu.make_async_copy(
                bufs[s], out_hbm.at[wid, ngroups - NSETS + s], sss[s]).wait()

    return gather_kernel


TBLK = 16384


@functools.cache
def _build_transpose(v, d):
    nblk = (v + TBLK - 1) // TBLK
    half = TBLK // 2

    def body(wt_ref, out_ref):
        wt = wt_ref[...]
        out_ref[...] = jnp.concatenate(
            [jnp.transpose(wt[:, :half], (1, 0)),
             jnp.transpose(wt[:, half:], (1, 0))], axis=1)

    return pl.pallas_call(
        body,
        grid=(nblk,),
        in_specs=[pl.BlockSpec((d, TBLK), lambda i: (0, i))],
        out_specs=pl.BlockSpec((half, 2 * d), lambda i: (i, 0)),
        out_shape=jax.ShapeDtypeStruct((nblk * half, 2 * d), jnp.float32),
    ), nblk * TBLK


def kernel(inp, weight):
    total = inp.shape[0] * inp.shape[1]
    nchunk = total // NW // CHUNK
    v, d = weight.shape
    xpose, vpad = _build_transpose(v, d)
    table = xpose(weight.T).reshape(vpad, d)
    t = inp.astype(jnp.int32)
    tj = t % TBLK
    ridx = (t - tj) + 2 * (tj % (TBLK // 2)) + tj // (TBLK // 2)
    idx = ridx.reshape(NW, nchunk, CHUNK)
    out = _build(total)(idx, table)
    return out.reshape(inp.shape[0], inp.shape[1], weight.shape[1])


# speedup vs baseline: 1.5488x; 1.0187x over previous
import functools

import jax
import jax.numpy as jnp
from jax import lax
from jax.experimental import pallas as pl
from jax.experimental.pallas import tpu as pltpu
from jax.experimental.pallas import tpu_sc as plsc

D = 64
NC = 2
NS = 16
NW = NC * NS
CHUNK = 128
K = 2
NSETS = 4
GROUP_ROWS = K * CHUNK


@functools.cache
def _build(total):
    per_w = total // NW
    nchunk = per_w // CHUNK
    ngroups = nchunk // K
    nround = ngroups // NSETS
    mesh = plsc.VectorSubcoreMesh(core_axis_name="c", subcore_axis_name="s")

    @functools.partial(
        pl.kernel,
        mesh=mesh,
        out_type=jax.ShapeDtypeStruct((NW, ngroups, GROUP_ROWS, D), jnp.float32),
        scratch_types=[
            pltpu.VMEM((nchunk, CHUNK), jnp.int32),
        ] + [pltpu.VMEM((GROUP_ROWS, D), jnp.float32)] * NSETS
          + [pltpu.SemaphoreType.DMA] * (2 * NSETS),
        compiler_params=pltpu.CompilerParams(use_tc_tiling_on_sc=False),
    )
    def gather_kernel(idx_hbm, table_hbm, out_hbm, idx_v, *bufsem):
        bufs = bufsem[:NSETS]
        sgs = bufsem[NSETS:2 * NSETS]
        sss = bufsem[2 * NSETS:]
        wid = lax.axis_index("s") * NC + lax.axis_index("c")
        pltpu.sync_copy(idx_hbm.at[wid], idx_v)

        def round_body(p, carry):
            copies = []
            for s in range(NSETS):
                g = NSETS * p + s
                @pl.when(p > 0)
                def _():
                    pltpu.make_async_copy(
                        bufs[s], out_hbm.at[wid, g - NSETS], sss[s]).wait()
                copies.append([
                    pltpu.async_copy(
                        table_hbm.at[idx_v.at[g * K + b]],
                        bufs[s].at[pl.ds(b * CHUNK, CHUNK)], sgs[s])
                    for b in range(K)
                ])
            for s in range(NSETS):
                g = NSETS * p + s
                for c in copies[s]:
                    c.wait()
                pltpu.async_copy(bufs[s], out_hbm.at[wid, g], sss[s])
            return carry

        lax.fori_loop(0, nround, round_body, 0)
        for s in range(NSETS):
            pltpu.make_async_copy(
                bufs[s], out_hbm.at[wid, ngroups - NSETS + s], sss[s]).wait()

    return gather_kernel


TBLK = 32768


@functools.cache
def _build_transpose(v, d):
    nblk = (v + TBLK - 1) // TBLK
    half = TBLK // 2

    def body(wt_ref, out_ref):
        wt = wt_ref[...]
        out_ref[...] = jnp.concatenate(
            [jnp.transpose(wt[:, :half], (1, 0)),
             jnp.transpose(wt[:, half:], (1, 0))], axis=1)

    return pl.pallas_call(
        body,
        grid=(nblk,),
        in_specs=[pl.BlockSpec((d, TBLK), lambda i: (0, i))],
        out_specs=pl.BlockSpec((half, 2 * d), lambda i: (i, 0)),
        out_shape=jax.ShapeDtypeStruct((nblk * half, 2 * d), jnp.float32),
    ), nblk * TBLK


def kernel(inp, weight):
    total = inp.shape[0] * inp.shape[1]
    nchunk = total // NW // CHUNK
    v, d = weight.shape
    xpose, vpad = _build_transpose(v, d)
    table = xpose(weight.T).reshape(vpad, d)
    t = inp.astype(jnp.int32)
    tj = t % TBLK
    ridx = (t - tj) + 2 * (tj % (TBLK // 2)) + tj // (TBLK // 2)
    idx = ridx.reshape(NW, nchunk, CHUNK)
    out = _build(total)(idx, table)
    return out.reshape(inp.shape[0], inp.shape[1], weight.shape[1])
